# baseline (speedup 1.0000x reference)
import functools

import jax
import jax.numpy as jnp
from jax import lax
from jax.experimental import pallas as pl
from jax.experimental.pallas import tpu as pltpu
from jax.experimental.pallas import tpu_sc as plsc

N = 10000
E = 320000
D = 128
DH = D // 2
LEAKY = 0.2

NS = 16
CH = 80
EPT = 20160
NCHUNK = EPT // CH
CPB = 18
NB = NCHUNK // CPB
NBUF = 3
SB = 624
REM = N - NS * SB
OB = 48


def _hgcn_body(rows_hbm, cols_hbm, vals_hbm, embs2_hbm, out2_hbm,
               h_sp, o_sp,
               rvA, cvA, vvA, rvB, cvB, vvB,
               ib0, ib1, ib2, gb0, gb1, gb2, sb0, sb1, sb2, obuf,
               gsem0, gsem1, gsem2, ssem0, ssem1, ssem2, stsemA, stsemB):
    c = lax.axis_index("c")
    s = lax.axis_index("s")
    ibuf = (ib0, ib1, ib2)
    gbuf = (gb0, gb1, gb2)
    sbuf = (sb0, sb1, sb2)
    gsem = (gsem0, gsem1, gsem2)
    ssem = (ssem0, ssem1, ssem2)
    stage = ((rvA, cvA, vvA, stsemA), (rvB, cvB, vvB, stsemB))

    def zbody(i, _):
        zero = jnp.zeros((16,), jnp.float32)
        for j in range(DH // 16):
            obuf[i, pl.ds(j * 16, 16)] = zero
        return 0
    lax.fori_loop(0, OB, zbody, 0)
    rbase = s * SB
    for k in range(SB // OB):
        pltpu.sync_copy(obuf, h_sp.at[pl.ds(rbase + k * OB, OB)])
        pltpu.sync_copy(obuf, o_sp.at[pl.ds(rbase + k * OB, OB)])
    @pl.when(s == NS - 1)
    def _():
        pltpu.sync_copy(obuf.at[pl.ds(0, REM)], h_sp.at[pl.ds(NS * SB, REM)])
        pltpu.sync_copy(obuf.at[pl.ds(0, REM)], o_sp.at[pl.ds(NS * SB, REM)])
    plsc.subcore_barrier()

    def _scale(gb, sb_, vv, q):
        for t in range(CH // 16):
            vvec = vv[q, pl.ds(t * 16, 16)]
            base = t * 16
            for lane in range(16):
                v = vvec[lane]
                for j in range(DH // 16):
                    sl = pl.ds(j * 16, 16)
                    sb_[base + lane, sl] = gb[base + lane, sl] * v

    def _stage_issue(blk, buf):
        rv, cv, vv, sem = buf
        pltpu.async_copy(rows_hbm.at[s, blk], rv, sem)
        pltpu.async_copy(cols_hbm.at[s, blk], cv, sem)
        pltpu.async_copy(vals_hbm.at[s, blk], vv, sem)

    def _stage_wait(buf):
        rv, cv, vv, sem = buf
        pltpu.make_async_copy(rows_hbm.at[s, 0], rv, sem).wait()
        pltpu.make_async_copy(cols_hbm.at[s, 0], cv, sem).wait()
        pltpu.make_async_copy(vals_hbm.at[s, 0], vv, sem).wait()

    def _hop(gather_issue, gather_wait, scat_ref, scat_sel):
        _stage_issue(0, stage[0])
        _stage_issue(1, stage[1])

        def _block(blk, buf, nxt):
            _stage_wait(buf)
            sidx = scat_sel(buf)
            for b in range(NBUF):
                gather_issue(buf, b, b)
            def body(t, _):
                for b in range(NBUF):
                    q = NBUF * t + b
                    gather_wait(b)
                    @pl.when(t > 0)
                    def _():
                        pltpu.make_async_copy(
                            sbuf[b], scat_ref.at[sidx.at[q]], ssem[b]).wait()
                    _scale(gbuf[b], sbuf[b], buf[2], q)
                    @pl.when(t < CPB // NBUF - 1)
                    def _():
                        gather_issue(buf, q + NBUF, b)
                    pltpu.async_copy(
                        sbuf[b], scat_ref.at[sidx.at[q]], ssem[b], add=True)
                return 0
            lax.fori_loop(0, CPB // NBUF, body, 0)
            for b in range(NBUF):
                q = CPB - NBUF + b
                pltpu.make_async_copy(
                    sbuf[b], scat_ref.at[sidx.at[q]], ssem[b]).wait()
            @pl.when(blk + 2 < NB)
            def _():
                _stage_issue(blk + 2, buf)

        def blkpair(bp, _):
            _block(2 * bp, stage[0], stage[1])
            _block(2 * bp + 1, stage[1], stage[0])
            return 0
        lax.fori_loop(0, NB // 2, blkpair, 0)

    def h1_issue(buf, q, b):
        for j in range(CH // 16):
            sl = pl.ds(j * 16, 16)
            ibuf[b][sl] = buf[0][q, sl] * 2 + c
        pltpu.async_copy(embs2_hbm.at[ibuf[b]], gbuf[b], gsem[b])
    def h1_wait(b):
        pltpu.make_async_copy(embs2_hbm.at[ibuf[b]], gbuf[b], gsem[b]).wait()
    _hop(h1_issue, h1_wait, h_sp, lambda buf: buf[1])
    plsc.subcore_barrier()

    def h2_issue(buf, q, b):
        pltpu.async_copy(h_sp.at[buf[1].at[q]], gbuf[b], gsem[b])
    def h2_wait(b):
        pltpu.make_async_copy(h_sp.at[stage[0][1].at[0]],
                              gbuf[b], gsem[b]).wait()
    _hop(h2_issue, h2_wait, o_sp, lambda buf: buf[0])
    plsc.subcore_barrier()

    def _leaky(nrows):
        def lbody(i, _):
            for j in range(DH // 16):
                sl = pl.ds(j * 16, 16)
                x = obuf[i, sl]
                obuf[i, sl] = jnp.where(x >= 0, x, x * LEAKY)
            return 0
        lax.fori_loop(0, nrows, lbody, 0)

    csl = pl.ds(c * DH, DH)
    for k in range(SB // OB):
        ro = rbase + k * OB
        pltpu.sync_copy(o_sp.at[pl.ds(ro, OB)], obuf)
        _leaky(OB)
        pltpu.sync_copy(obuf, out2_hbm.at[pl.ds(ro, OB), csl])
    @pl.when(s == NS - 1)
    def _():
        pltpu.sync_copy(o_sp.at[pl.ds(NS * SB, REM)], obuf.at[pl.ds(0, REM)])
        _leaky(REM)
        pltpu.sync_copy(obuf.at[pl.ds(0, REM)],
                        out2_hbm.at[pl.ds(NS * SB, REM), csl])


@jax.jit
def _hgcn_sc(rows, cols, vals, embs2):
    mesh = plsc.VectorSubcoreMesh(core_axis_name="c", subcore_axis_name="s")
    return pl.kernel(
        _hgcn_body,
        out_type=jax.ShapeDtypeStruct((N, D), jnp.float32),
        mesh=mesh,
        scratch_types=[
            pltpu.VMEM_SHARED((N, DH), jnp.float32),
            pltpu.VMEM_SHARED((N, DH), jnp.float32),
            pltpu.VMEM((CPB, CH), jnp.int32),
            pltpu.VMEM((CPB, CH), jnp.int32),
            pltpu.VMEM((CPB, CH), jnp.float32),
            pltpu.VMEM((CPB, CH), jnp.int32),
            pltpu.VMEM((CPB, CH), jnp.int32),
            pltpu.VMEM((CPB, CH), jnp.float32),
            pltpu.VMEM((CH,), jnp.int32),
            pltpu.VMEM((CH,), jnp.int32),
            pltpu.VMEM((CH,), jnp.int32),
            pltpu.VMEM((CH, DH), jnp.float32),
            pltpu.VMEM((CH, DH), jnp.float32),
            pltpu.VMEM((CH, DH), jnp.float32),
            pltpu.VMEM((CH, DH), jnp.float32),
            pltpu.VMEM((CH, DH), jnp.float32),
            pltpu.VMEM((CH, DH), jnp.float32),
            pltpu.VMEM((OB, DH), jnp.float32),
            pltpu.SemaphoreType.DMA,
            pltpu.SemaphoreType.DMA,
            pltpu.SemaphoreType.DMA,
            pltpu.SemaphoreType.DMA,
            pltpu.SemaphoreType.DMA,
            pltpu.SemaphoreType.DMA,
            pltpu.SemaphoreType.DMA,
            pltpu.SemaphoreType.DMA,
        ],
        compiler_params=pltpu.CompilerParams(use_tc_tiling_on_sc=False),
    )(rows, cols, vals, embs2)


def _pad_tile(x):
    x = x.reshape(NS, E // NS)
    x = jnp.pad(x, ((0, 0), (0, EPT - E // NS)))
    return x.reshape(NS, NB, CPB, CH)


def kernel(edge_index, values, embs):
    rows = _pad_tile(edge_index[0].astype(jnp.int32))
    cols = _pad_tile(edge_index[1].astype(jnp.int32))
    vals = _pad_tile(values)
    embs2 = embs.reshape(2 * N, DH)
    return _hgcn_sc(rows, cols, vals, embs2)


# speedup vs baseline: 1.0735x; 1.0735x over previous
import functools

import jax
import jax.numpy as jnp
from jax import lax
from jax.experimental import pallas as pl
from jax.experimental.pallas import tpu as pltpu
from jax.experimental.pallas import tpu_sc as plsc

N = 10000
E = 320000
D = 128
DH = D // 2
LEAKY = 0.2

NS = 16
CH = 80
EPT = 20160
NCHUNK = EPT // CH
CPB = 36
NB = NCHUNK // CPB
NBUF = 3
SB = 624
REM = N - NS * SB
OB = 48


def _hgcn_body(rows_hbm, cols_hbm, vals_hbm, embs2_hbm, out2_hbm,
               h_sp, o_sp,
               rvA, cvA, vvA, rvB, cvB, vvB,
               ib0, ib1, ib2, gb0, gb1, gb2, sb0, sb1, sb2, obuf,
               gsem0, gsem1, gsem2, ssem0, ssem1, ssem2, stsemA, stsemB):
    c = lax.axis_index("c")
    s = lax.axis_index("s")
    ibuf = (ib0, ib1, ib2)
    gbuf = (gb0, gb1, gb2)
    sbuf = (sb0, sb1, sb2)
    gsem = (gsem0, gsem1, gsem2)
    ssem = (ssem0, ssem1, ssem2)
    stage = ((rvA, cvA, vvA, stsemA), (rvB, cvB, vvB, stsemB))

    def zbody(i, _):
        zero = jnp.zeros((16,), jnp.float32)
        for j in range(DH // 16):
            obuf[i, pl.ds(j * 16, 16)] = zero
        return 0
    lax.fori_loop(0, OB, zbody, 0)
    rbase = s * SB
    for k in range(SB // OB):
        pltpu.sync_copy(obuf, h_sp.at[pl.ds(rbase + k * OB, OB)])
        pltpu.sync_copy(obuf, o_sp.at[pl.ds(rbase + k * OB, OB)])
    @pl.when(s == NS - 1)
    def _():
        pltpu.sync_copy(obuf.at[pl.ds(0, REM)], h_sp.at[pl.ds(NS * SB, REM)])
        pltpu.sync_copy(obuf.at[pl.ds(0, REM)], o_sp.at[pl.ds(NS * SB, REM)])
    plsc.subcore_barrier()

    def _scale(gb, sb_, vv, q):
        for t in range(CH // 16):
            vvec = vv[q, pl.ds(t * 16, 16)]
            base = t * 16
            for lane in range(16):
                v = vvec[lane]
                for j in range(DH // 16):
                    sl = pl.ds(j * 16, 16)
                    sb_[base + lane, sl] = gb[base + lane, sl] * v

    def _stage_issue(blk, buf):
        rv, cv, vv, sem = buf
        pltpu.async_copy(rows_hbm.at[s, blk], rv, sem)
        pltpu.async_copy(cols_hbm.at[s, blk], cv, sem)
        pltpu.async_copy(vals_hbm.at[s, blk], vv, sem)

    def _stage_wait(buf):
        rv, cv, vv, sem = buf
        pltpu.make_async_copy(rows_hbm.at[s, 0], rv, sem).wait()
        pltpu.make_async_copy(cols_hbm.at[s, 0], cv, sem).wait()
        pltpu.make_async_copy(vals_hbm.at[s, 0], vv, sem).wait()

    def _hop(gather_issue, gather_wait, scat_ref, scat_sel):
        def _block(blk, buf, nxt):
            _stage_issue(blk, buf)
            _stage_wait(buf)
            sidx = scat_sel(buf)
            for b in range(NBUF):
                gather_issue(buf, b, b)
            def body(t, _):
                for b in range(NBUF):
                    q = NBUF * t + b
                    gather_wait(b)
                    @pl.when(t > 0)
                    def _():
                        pltpu.make_async_copy(
                            sbuf[b], scat_ref.at[sidx.at[q]], ssem[b]).wait()
                    _scale(gbuf[b], sbuf[b], buf[2], q)
                    @pl.when(t < CPB // NBUF - 1)
                    def _():
                        gather_issue(buf, q + NBUF, b)
                    pltpu.async_copy(
                        sbuf[b], scat_ref.at[sidx.at[q]], ssem[b], add=True)
                return 0
            lax.fori_loop(0, CPB // NBUF, body, 0)
            for b in range(NBUF):
                q = CPB - NBUF + b
                pltpu.make_async_copy(
                    sbuf[b], scat_ref.at[sidx.at[q]], ssem[b]).wait()

        def blkbody(blk, _):
            _block(blk, stage[0], stage[1])
            return 0
        lax.fori_loop(0, NB, blkbody, 0)

    def h1_issue(buf, q, b):
        for j in range(CH // 16):
            sl = pl.ds(j * 16, 16)
            ibuf[b][sl] = buf[0][q, sl] * 2 + c
        pltpu.async_copy(embs2_hbm.at[ibuf[b]], gbuf[b], gsem[b])
    def h1_wait(b):
        pltpu.make_async_copy(embs2_hbm.at[ibuf[b]], gbuf[b], gsem[b]).wait()
    _hop(h1_issue, h1_wait, h_sp, lambda buf: buf[1])
    plsc.subcore_barrier()

    def h2_issue(buf, q, b):
        pltpu.async_copy(h_sp.at[buf[1].at[q]], gbuf[b], gsem[b])
    def h2_wait(b):
        pltpu.make_async_copy(h_sp.at[stage[0][1].at[0]],
                              gbuf[b], gsem[b]).wait()
    _hop(h2_issue, h2_wait, o_sp, lambda buf: buf[0])
    plsc.subcore_barrier()

    def _leaky(nrows):
        def lbody(i, _):
            for j in range(DH // 16):
                sl = pl.ds(j * 16, 16)
                x = obuf[i, sl]
                obuf[i, sl] = jnp.where(x >= 0, x, x * LEAKY)
            return 0
        lax.fori_loop(0, nrows, lbody, 0)

    csl = pl.ds(c * DH, DH)
    for k in range(SB // OB):
        ro = rbase + k * OB
        pltpu.sync_copy(o_sp.at[pl.ds(ro, OB)], obuf)
        _leaky(OB)
        pltpu.sync_copy(obuf, out2_hbm.at[pl.ds(ro, OB), csl])
    @pl.when(s == NS - 1)
    def _():
        pltpu.sync_copy(o_sp.at[pl.ds(NS * SB, REM)], obuf.at[pl.ds(0, REM)])
        _leaky(REM)
        pltpu.sync_copy(obuf.at[pl.ds(0, REM)],
                        out2_hbm.at[pl.ds(NS * SB, REM), csl])


@jax.jit
def _hgcn_sc(rows, cols, vals, embs2):
    mesh = plsc.VectorSubcoreMesh(core_axis_name="c", subcore_axis_name="s")
    return pl.kernel(
        _hgcn_body,
        out_type=jax.ShapeDtypeStruct((N, D), jnp.float32),
        mesh=mesh,
        scratch_types=[
            pltpu.VMEM_SHARED((N, DH), jnp.float32),
            pltpu.VMEM_SHARED((N, DH), jnp.float32),
            pltpu.VMEM((CPB, CH), jnp.int32),
            pltpu.VMEM((CPB, CH), jnp.int32),
            pltpu.VMEM((CPB, CH), jnp.float32),
            pltpu.VMEM((CPB, CH), jnp.int32),
            pltpu.VMEM((CPB, CH), jnp.int32),
            pltpu.VMEM((CPB, CH), jnp.float32),
            pltpu.VMEM((CH,), jnp.int32),
            pltpu.VMEM((CH,), jnp.int32),
            pltpu.VMEM((CH,), jnp.int32),
            pltpu.VMEM((CH, DH), jnp.float32),
            pltpu.VMEM((CH, DH), jnp.float32),
            pltpu.VMEM((CH, DH), jnp.float32),
            pltpu.VMEM((CH, DH), jnp.float32),
            pltpu.VMEM((CH, DH), jnp.float32),
            pltpu.VMEM((CH, DH), jnp.float32),
            pltpu.VMEM((OB, DH), jnp.float32),
            pltpu.SemaphoreType.DMA,
            pltpu.SemaphoreType.DMA,
            pltpu.SemaphoreType.DMA,
            pltpu.SemaphoreType.DMA,
            pltpu.SemaphoreType.DMA,
            pltpu.SemaphoreType.DMA,
            pltpu.SemaphoreType.DMA,
            pltpu.SemaphoreType.DMA,
        ],
        compiler_params=pltpu.CompilerParams(use_tc_tiling_on_sc=False),
    )(rows, cols, vals, embs2)


def _pad_tile(x):
    x = x.reshape(NS, E // NS)
    x = jnp.pad(x, ((0, 0), (0, EPT - E // NS)))
    return x.reshape(NS, NB, CPB, CH)


def kernel(edge_index, values, embs):
    rows = _pad_tile(edge_index[0].astype(jnp.int32))
    cols = _pad_tile(edge_index[1].astype(jnp.int32))
    vals = _pad_tile(values)
    embs2 = embs.reshape(2 * N, DH)
    return _hgcn_sc(rows, cols, vals, embs2)


# speedup vs baseline: 1.0762x; 1.0026x over previous
import functools

import jax
import jax.numpy as jnp
from jax import lax
from jax.experimental import pallas as pl
from jax.experimental.pallas import tpu as pltpu
from jax.experimental.pallas import tpu_sc as plsc

N = 10000
E = 320000
D = 128
DH = D // 2
LEAKY = 0.2

NS = 16
CH = 80
EPT = E // NS
NCHUNK = EPT // CH
CPB = 50
NB = NCHUNK // CPB
SB = 624
REM = N - NS * SB
OB = 48


def _hgcn_body(rows_hbm, cols_hbm, vals_hbm, embs2_hbm,
               out2_hbm, h2_hbm,
               h_sp, o_sp,
               rows_vm, cols_vm, vals_vm,
               ib0, ib1, gb0, gb1, sb0, sb1, obuf,
               gsem0, gsem1, ssem0, ssem1):
    c = lax.axis_index("c")
    s = lax.axis_index("s")
    cN = c * N
    ibuf = (ib0, ib1)
    gbuf = (gb0, gb1)
    sbuf = (sb0, sb1)
    gsem = (gsem0, gsem1)
    ssem = (ssem0, ssem1)
    csl = pl.ds(c * DH, DH)

    def _zero_obuf():
        def zbody(i, _):
            zero = jnp.zeros((16,), jnp.float32)
            for j in range(DH // 16):
                obuf[i, pl.ds(j * 16, 16)] = zero
            return 0
        lax.fori_loop(0, OB, zbody, 0)

    def _zero_acc(acc):
        for k in range(SB // OB):
            pltpu.sync_copy(obuf, acc.at[pl.ds(rbase + k * OB, OB)])
        @pl.when(s == NS - 1)
        def _():
            pltpu.sync_copy(obuf.at[pl.ds(0, REM)],
                            acc.at[pl.ds(NS * SB, REM)])

    _zero_obuf()
    rbase = s * SB
    _zero_acc(h_sp)
    pltpu.sync_copy(embs2_hbm.at[pl.ds(cN + rbase, SB)],
                    o_sp.at[pl.ds(rbase, SB)])
    @pl.when(s == NS - 1)
    def _():
        pltpu.sync_copy(embs2_hbm.at[pl.ds(cN + NS * SB, REM)],
                        o_sp.at[pl.ds(NS * SB, REM)])
    plsc.subcore_barrier()

    def _scale(gb, sb_, q):
        for t in range(CH // 16):
            vvec = vals_vm[q, pl.ds(t * 16, 16)]
            base = t * 16
            for lane in range(16):
                v = vvec[lane]
                for j in range(DH // 16):
                    sl = pl.ds(j * 16, 16)
                    sb_[base + lane, sl] = gb[base + lane, sl] * v

    def _hop(gather_issue, gather_wait, scat_ref, scat_vm):
        def blk_body(blk, _):
            pltpu.sync_copy(rows_hbm.at[s, blk], rows_vm)
            pltpu.sync_copy(cols_hbm.at[s, blk], cols_vm)
            pltpu.sync_copy(vals_hbm.at[s, blk], vals_vm)
            for b in (0, 1):
                gather_issue(b, b)
            def body(t, _):
                for b in (0, 1):
                    q = 2 * t + b
                    gather_wait(b)
                    @pl.when(t > 0)
                    def _():
                        pltpu.make_async_copy(
                            sbuf[b], scat_ref.at[scat_vm.at[q]],
                            ssem[b]).wait()
                    _scale(gbuf[b], sbuf[b], q)
                    pltpu.async_copy(
                        sbuf[b], scat_ref.at[scat_vm.at[q]], ssem[b],
                        add=True)
                    @pl.when(t < CPB // 2 - 1)
                    def _():
                        gather_issue(q + 2, b)
                return 0
            lax.fori_loop(0, CPB // 2, body, 0)
            for b in (0, 1):
                q = CPB - 2 + b
                pltpu.make_async_copy(
                    sbuf[b], scat_ref.at[scat_vm.at[q]], ssem[b]).wait()
            return 0
        lax.fori_loop(0, NB, blk_body, 0)

    def h1_issue(q, b):
        if b == 0:
            for j in range(CH // 16):
                sl = pl.ds(j * 16, 16)
                ibuf[0][sl] = rows_vm[q, sl] + cN
            pltpu.async_copy(embs2_hbm.at[ibuf[0]], gbuf[0], gsem[0])
        else:
            pltpu.async_copy(o_sp.at[rows_vm.at[q]], gbuf[1], gsem[1])
    def h1_wait(b):
        if b == 0:
            pltpu.make_async_copy(embs2_hbm.at[ibuf[0]],
                                  gbuf[0], gsem[0]).wait()
        else:
            pltpu.make_async_copy(o_sp.at[rows_vm.at[0]],
                                  gbuf[1], gsem[1]).wait()
    _hop(h1_issue, h1_wait, h_sp, cols_vm)
    plsc.subcore_barrier()

    pltpu.sync_copy(h_sp.at[pl.ds(rbase, SB)],
                    h2_hbm.at[pl.ds(cN + rbase, SB)])
    @pl.when(s == NS - 1)
    def _():
        pltpu.sync_copy(h_sp.at[pl.ds(NS * SB, REM)],
                        h2_hbm.at[pl.ds(cN + NS * SB, REM)])
    _zero_obuf()
    _zero_acc(o_sp)
    plsc.subcore_barrier()

    def h2_issue(q, b):
        if b == 0:
            for j in range(CH // 16):
                sl = pl.ds(j * 16, 16)
                ibuf[0][sl] = cols_vm[q, sl] + cN
            pltpu.async_copy(h2_hbm.at[ibuf[0]], gbuf[0], gsem[0])
        else:
            pltpu.async_copy(h_sp.at[cols_vm.at[q]], gbuf[1], gsem[1])
    def h2_wait(b):
        if b == 0:
            pltpu.make_async_copy(h2_hbm.at[ibuf[0]],
                                  gbuf[0], gsem[0]).wait()
        else:
            pltpu.make_async_copy(h_sp.at[cols_vm.at[0]],
                                  gbuf[1], gsem[1]).wait()
    _hop(h2_issue, h2_wait, o_sp, rows_vm)
    plsc.subcore_barrier()

    def _leaky(nrows):
        def lbody(i, _):
            for j in range(DH // 16):
                sl = pl.ds(j * 16, 16)
                x = obuf[i, sl]
                obuf[i, sl] = jnp.where(x >= 0, x, x * LEAKY)
            return 0
        lax.fori_loop(0, nrows, lbody, 0)

    for k in range(SB // OB):
        ro = rbase + k * OB
        pltpu.sync_copy(o_sp.at[pl.ds(ro, OB)], obuf)
        _leaky(OB)
        pltpu.sync_copy(obuf, out2_hbm.at[pl.ds(ro, OB), csl])
    @pl.when(s == NS - 1)
    def _():
        pltpu.sync_copy(o_sp.at[pl.ds(NS * SB, REM)], obuf.at[pl.ds(0, REM)])
        _leaky(REM)
        pltpu.sync_copy(obuf.at[pl.ds(0, REM)],
                        out2_hbm.at[pl.ds(NS * SB, REM), csl])


@jax.jit
def _hgcn_sc(rows, cols, vals, embs2):
    mesh = plsc.VectorSubcoreMesh(core_axis_name="c", subcore_axis_name="s")
    return pl.kernel(
        _hgcn_body,
        out_type=(jax.ShapeDtypeStruct((N, D), jnp.float32),
                  jax.ShapeDtypeStruct((2 * N, DH), jnp.float32)),
        mesh=mesh,
        scratch_types=[
            pltpu.VMEM_SHARED((N, DH), jnp.float32),
            pltpu.VMEM_SHARED((N, DH), jnp.float32),
            pltpu.VMEM((CPB, CH), jnp.int32),
            pltpu.VMEM((CPB, CH), jnp.int32),
            pltpu.VMEM((CPB, CH), jnp.float32),
            pltpu.VMEM((CH,), jnp.int32),
            pltpu.VMEM((CH,), jnp.int32),
            pltpu.VMEM((CH, DH), jnp.float32),
            pltpu.VMEM((CH, DH), jnp.float32),
            pltpu.VMEM((CH, DH), jnp.float32),
            pltpu.VMEM((CH, DH), jnp.float32),
            pltpu.VMEM((OB, DH), jnp.float32),
            pltpu.SemaphoreType.DMA,
            pltpu.SemaphoreType.DMA,
            pltpu.SemaphoreType.DMA,
            pltpu.SemaphoreType.DMA,
        ],
        compiler_params=pltpu.CompilerParams(use_tc_tiling_on_sc=False),
    )(rows, cols, vals, embs2)


def kernel(edge_index, values, embs):
    rows = edge_index[0].astype(jnp.int32).reshape(NS, NB, CPB, CH)
    cols = edge_index[1].astype(jnp.int32).reshape(NS, NB, CPB, CH)
    vals = values.reshape(NS, NB, CPB, CH)
    embs2 = jnp.concatenate([embs[:, :DH], embs[:, DH:]], axis=0)
    out, _ = _hgcn_sc(rows, cols, vals, embs2)
    return out


# speedup vs baseline: 1.2469x; 1.1586x over previous
import functools

import jax
import jax.numpy as jnp
from jax import lax
from jax.experimental import pallas as pl
from jax.experimental.pallas import tpu as pltpu
from jax.experimental.pallas import tpu_sc as plsc

N = 10000
E = 320000
D = 128
DH = D // 2
LEAKY = 0.2

NS = 16
CH = 80
EPT = E // NS
NCHUNK = EPT // CH
CPB = 50
NB = NCHUNK // CPB
SB = 624
REM = N - NS * SB
OB = 48


HA = 48
HB = CH - HA


def _hgcn_body(rows_hbm, cols_hbm, vals_hbm, embs2_hbm, out2_hbm,
               h_sp, o_sp,
               rows_vm, cols_vm, vals_vm,
               ib0, ib1, sa0, sa1, sb0i, sb1i, gb0, gb1, sb0, sb1, obuf,
               gsem0, gsem1, ssem0, ssem1):
    c = lax.axis_index("c")
    s = lax.axis_index("s")
    cN = c * N
    ibuf = (ib0, ib1)
    sidxa = (sa0, sa1)
    sidxb = (sb0i, sb1i)
    gbuf = (gb0, gb1)
    sbuf = (sb0, sb1)
    gsem = (gsem0, gsem1)
    ssem = (ssem0, ssem1)

    def zbody(i, _):
        zero = jnp.zeros((16,), jnp.float32)
        for j in range(DH // 16):
            obuf[i, pl.ds(j * 16, 16)] = zero
        return 0
    lax.fori_loop(0, OB, zbody, 0)
    rbase = s * SB
    for k in range(SB // OB):
        pltpu.sync_copy(obuf, h_sp.at[pl.ds(rbase + k * OB, OB)])
        pltpu.sync_copy(obuf, o_sp.at[pl.ds(rbase + k * OB, OB)])
    @pl.when(s == NS - 1)
    def _():
        pltpu.sync_copy(obuf.at[pl.ds(0, REM)], h_sp.at[pl.ds(NS * SB, REM)])
        pltpu.sync_copy(obuf.at[pl.ds(0, REM)], o_sp.at[pl.ds(NS * SB, REM)])
    plsc.subcore_barrier()

    def _scale(gb, sb_, q, r0, nr):
        for t in range(r0 // 16, (r0 + nr) // 16):
            vvec = vals_vm[q, pl.ds(t * 16, 16)]
            base = t * 16
            for lane in range(16):
                v = vvec[lane]
                for j in range(DH // 16):
                    sl = pl.ds(j * 16, 16)
                    sb_[base + lane, sl] = gb[base + lane, sl] * v

    def _drain_scat(b, scat_ref):
        pltpu.make_async_copy(
            sbuf[b].at[pl.ds(0, HA)], scat_ref.at[sidxa[b]], ssem[b]).wait()
        pltpu.make_async_copy(
            sbuf[b].at[pl.ds(HA, HB)], scat_ref.at[sidxb[b]], ssem[b]).wait()

    def _hop(gather_issue, gather_wait, scat_ref, scat_vm):
        def blk_body(blk, _):
            pltpu.sync_copy(rows_hbm.at[s, blk], rows_vm)
            pltpu.sync_copy(cols_hbm.at[s, blk], cols_vm)
            pltpu.sync_copy(vals_hbm.at[s, blk], vals_vm)
            for b in (0, 1):
                gather_issue(b, b)
            def body(t, _):
                for b in (0, 1):
                    q = 2 * t + b
                    gather_wait(b)
                    @pl.when(t > 0)
                    def _():
                        _drain_scat(b, scat_ref)
                    for j in range(HA // 16):
                        sl = pl.ds(j * 16, 16)
                        sidxa[b][sl] = scat_vm[q, sl]
                    for j in range(HB // 16):
                        sl = pl.ds(j * 16, 16)
                        sidxb[b][sl] = scat_vm[q, pl.ds(HA + j * 16, 16)]
                    _scale(gbuf[b], sbuf[b], q, 0, HA)
                    pltpu.async_copy(
                        sbuf[b].at[pl.ds(0, HA)], scat_ref.at[sidxa[b]],
                        ssem[b], add=True)
                    _scale(gbuf[b], sbuf[b], q, HA, HB)
                    pltpu.async_copy(
                        sbuf[b].at[pl.ds(HA, HB)], scat_ref.at[sidxb[b]],
                        ssem[b], add=True)
                    @pl.when(t < CPB // 2 - 1)
                    def _():
                        gather_issue(q + 2, b)
                return 0
            lax.fori_loop(0, CPB // 2, body, 0)
            for b in (0, 1):
                _drain_scat(b, scat_ref)
            return 0
        lax.fori_loop(0, NB, blk_body, 0)

    def h1_issue(q, b):
        for j in range(CH // 16):
            sl = pl.ds(j * 16, 16)
            ibuf[b][sl] = rows_vm[q, sl] * 2 + c
        pltpu.async_copy(embs2_hbm.at[ibuf[b]], gbuf[b], gsem[b])
    def h1_wait(b):
        pltpu.make_async_copy(embs2_hbm.at[ibuf[b]], gbuf[b], gsem[b]).wait()
    _hop(h1_issue, h1_wait, h_sp, cols_vm)
    plsc.subcore_barrier()

    def h2_issue(q, b):
        pltpu.async_copy(h_sp.at[cols_vm.at[q]], gbuf[b], gsem[b])
    def h2_wait(b):
        pltpu.make_async_copy(h_sp.at[cols_vm.at[0]], gbuf[b], gsem[b]).wait()
    _hop(h2_issue, h2_wait, o_sp, rows_vm)
    plsc.subcore_barrier()

    def _leaky(nrows):
        def lbody(i, _):
            for j in range(DH // 16):
                sl = pl.ds(j * 16, 16)
                x = obuf[i, sl]
                obuf[i, sl] = jnp.where(x >= 0, x, x * LEAKY)
            return 0
        lax.fori_loop(0, nrows, lbody, 0)

    csl = pl.ds(c * DH, DH)
    for k in range(SB // OB):
        ro = rbase + k * OB
        pltpu.sync_copy(o_sp.at[pl.ds(ro, OB)], obuf)
        _leaky(OB)
        pltpu.sync_copy(obuf, out2_hbm.at[pl.ds(ro, OB), csl])
    @pl.when(s == NS - 1)
    def _():
        pltpu.sync_copy(o_sp.at[pl.ds(NS * SB, REM)], obuf.at[pl.ds(0, REM)])
        _leaky(REM)
        pltpu.sync_copy(obuf.at[pl.ds(0, REM)],
                        out2_hbm.at[pl.ds(NS * SB, REM), csl])


@jax.jit
def _hgcn_sc(rows, cols, vals, embs2):
    mesh = plsc.VectorSubcoreMesh(core_axis_name="c", subcore_axis_name="s")
    return pl.kernel(
        _hgcn_body,
        out_type=jax.ShapeDtypeStruct((N, D), jnp.float32),
        mesh=mesh,
        scratch_types=[
            pltpu.VMEM_SHARED((N, DH), jnp.float32),
            pltpu.VMEM_SHARED((N, DH), jnp.float32),
            pltpu.VMEM((CPB, CH), jnp.int32),
            pltpu.VMEM((CPB, CH), jnp.int32),
            pltpu.VMEM((CPB, CH), jnp.float32),
            pltpu.VMEM((CH,), jnp.int32),
            pltpu.VMEM((CH,), jnp.int32),
            pltpu.VMEM((HA,), jnp.int32),
            pltpu.VMEM((HA,), jnp.int32),
            pltpu.VMEM((HB,), jnp.int32),
            pltpu.VMEM((HB,), jnp.int32),
            pltpu.VMEM((CH, DH), jnp.float32),
            pltpu.VMEM((CH, DH), jnp.float32),
            pltpu.VMEM((CH, DH), jnp.float32),
            pltpu.VMEM((CH, DH), jnp.float32),
            pltpu.VMEM((OB, DH), jnp.float32),
            pltpu.SemaphoreType.DMA,
            pltpu.SemaphoreType.DMA,
            pltpu.SemaphoreType.DMA,
            pltpu.SemaphoreType.DMA,
        ],
        compiler_params=pltpu.CompilerParams(use_tc_tiling_on_sc=False),
    )(rows, cols, vals, embs2)


def kernel(edge_index, values, embs):
    rows = edge_index[0].astype(jnp.int32).reshape(NS, NB, CPB, CH)
    cols = edge_index[1].astype(jnp.int32).reshape(NS, NB, CPB, CH)
    vals = values.reshape(NS, NB, CPB, CH)
    embs2 = embs.reshape(2 * N, DH)
    return _hgcn_sc(rows, cols, vals, embs2)
